# Initial kernel scaffold; baseline (speedup 1.0000x reference)
import jax
import jax.numpy as jnp
from jax.experimental import pallas as pl


def kernel(x, adj, W0_1, W1_1, b1, W0_2, W1_2, b2):
    raise NotImplementedError("write your pallas kernel here")


# speedup vs baseline: 12.0660x; 12.0660x over previous
import functools

import jax
import jax.numpy as jnp
from jax import lax
from jax.experimental import pallas as pl
from jax.experimental.pallas import tpu as pltpu
from jax.experimental.pallas import tpu_sc as plsc

N = 10000
E = 320000
F_IN = 128
HID = 64
N_CLASSES = 40

NC = 2
NS = 16
NW = NC * NS
NP = 10240
TILE_N = NP // NS
K = 128
CH = 80
EPT = CH * K
EPAD = EPT * NW

ROWS_PER_BOUNCE = 128
N_BOUNCE = TILE_N // ROWS_PER_BOUNCE

@functools.cache
def _get_mesh():
    return plsc.VectorSubcoreMesh(core_axis_name="c", subcore_axis_name="s",
                                  num_cores=NC, num_subcores=NS)


@functools.cache
def _get_deg_kernel():
    return functools.partial(
        pl.kernel,
        out_type=jax.ShapeDtypeStruct((NC, NP), jnp.float32),
        mesh=_get_mesh(),
        scratch_types=[
            pltpu.VMEM((CH, K), jnp.int32),
            pltpu.VMEM((K,), jnp.float32),
            pltpu.VMEM((TILE_N,), jnp.float32),
            pltpu.VMEM_SHARED((NP,), jnp.float32),
        ],
    )(_deg_body)


def _deg_body(row_hbm, out_hbm, rowv, ones, zbuf, acc):
    cid = lax.axis_index("c")
    sid = lax.axis_index("s")
    wid = sid * NC + cid

    pltpu.sync_copy(row_hbm.at[wid], rowv)

    def fill_ones(i, _):
        ones[pl.ds(i * 16, 16)] = jnp.full((16,), 1.0, jnp.float32)
        return 0

    lax.fori_loop(0, K // 16, fill_ones, 0)

    def fill_zero(i, _):
        zbuf[pl.ds(i * 16, 16)] = jnp.zeros((16,), jnp.float32)
        return 0

    lax.fori_loop(0, TILE_N // 16, fill_zero, 0)

    pltpu.sync_copy(zbuf, acc.at[pl.ds(sid * TILE_N, TILE_N)])
    plsc.subcore_barrier()

    def chunk(ci, _):
        pltpu.sync_copy(ones, acc.at[rowv.at[ci]], add=True)
        return 0

    lax.fori_loop(0, CH, chunk, 0)
    plsc.subcore_barrier()

    sl = pl.ds(sid * TILE_N, TILE_N)
    pltpu.sync_copy(acc.at[sl], zbuf)
    pltpu.sync_copy(zbuf, out_hbm.at[cid, sl])


@functools.cache
def _make_prop(D):
    @functools.partial(
        pl.kernel,
        out_type=jax.ShapeDtypeStruct((NC, NP, D), jnp.float32),
        mesh=_get_mesh(),
        scratch_types=[
            pltpu.VMEM((CH, K), jnp.int32),
            pltpu.VMEM((CH, K), jnp.int32),
            pltpu.VMEM((K, D), jnp.float32),
            pltpu.VMEM((ROWS_PER_BOUNCE, D), jnp.float32),
            pltpu.VMEM_SHARED((NP, D), jnp.float32),
            pltpu.SemaphoreType.DMA,
        ],
        compiler_params=pltpu.CompilerParams(use_tc_tiling_on_sc=False),
    )
    def _prop(y_hbm, row_hbm, col_hbm, out_hbm, rowv, colv, buf, zbuf, acc,
              gsem):
        cid = lax.axis_index("c")
        sid = lax.axis_index("s")
        wid = sid * NC + cid

        pltpu.sync_copy(row_hbm.at[wid], rowv)
        pltpu.sync_copy(col_hbm.at[wid], colv)

        def fill_zero(i, _):
            r = i // (D // 16)
            c = (i % (D // 16)) * 16
            zbuf[r, pl.ds(c, 16)] = jnp.zeros((16,), jnp.float32)
            return 0

        lax.fori_loop(0, ROWS_PER_BOUNCE * (D // 16), fill_zero, 0)

        for b in range(N_BOUNCE):
            pltpu.sync_copy(
                zbuf,
                acc.at[pl.ds(sid * TILE_N + b * ROWS_PER_BOUNCE,
                             ROWS_PER_BOUNCE), :])
        plsc.subcore_barrier()

        def chunk(ci, _):
            pltpu.async_copy(y_hbm.at[rowv.at[ci]], buf, gsem).wait()
            pltpu.sync_copy(buf, acc.at[colv.at[ci]], add=True)
            return 0

        lax.fori_loop(0, CH, chunk, 0)
        plsc.subcore_barrier()

        for b in range(N_BOUNCE):
            sl = pl.ds(sid * TILE_N + b * ROWS_PER_BOUNCE, ROWS_PER_BOUNCE)
            pltpu.sync_copy(acc.at[sl, :], zbuf)
            pltpu.sync_copy(zbuf, out_hbm.at[cid].at[sl, :])

    return _prop


_R = 1024
_G = NP // _R


def _tc_a_body(x_ref, w0_ref, w1_ref, b1_ref, deg_ref,
               d0_ref, y1_ref, dis_ref):
    deg = deg_ref[0, :] + deg_ref[1, :]
    dis = jnp.where(deg > 0, lax.rsqrt(deg), 0.0)[:, None]
    xb = x_ref[...]
    d0_ref[...] = jnp.dot(xb, w0_ref[...],
                          preferred_element_type=jnp.float32) + b1_ref[...]
    y1_ref[...] = jnp.dot(xb, w1_ref[...],
                          preferred_element_type=jnp.float32) * dis
    dis_ref[...] = dis


def _tc_a(x_p, W0_1, W1_1, b1, degp):
    return pl.pallas_call(
        _tc_a_body,
        grid=(_G,),
        in_specs=[
            pl.BlockSpec((_R, F_IN), lambda j: (j, 0)),
            pl.BlockSpec((F_IN, HID), lambda j: (0, 0)),
            pl.BlockSpec((F_IN, HID), lambda j: (0, 0)),
            pl.BlockSpec((1, HID), lambda j: (0, 0)),
            pl.BlockSpec((NC, _R), lambda j: (0, j)),
        ],
        out_specs=[
            pl.BlockSpec((_R, HID), lambda j: (j, 0)),
            pl.BlockSpec((_R, HID), lambda j: (j, 0)),
            pl.BlockSpec((_R, 1), lambda j: (j, 0)),
        ],
        out_shape=[
            jax.ShapeDtypeStruct((NP, HID), jnp.float32),
            jax.ShapeDtypeStruct((NP, HID), jnp.float32),
            jax.ShapeDtypeStruct((NP, 1), jnp.float32),
        ],
    )(x_p, W0_1, W1_1, b1, degp)


def _tc_b_body(d0_ref, p1_ref, dis_ref, w02_ref, w12_ref, b2_ref,
               d1_ref, y2_ref):
    p = p1_ref[0] + p1_ref[1]
    dis = dis_ref[...]
    h = jnp.maximum(d0_ref[...] - dis * p, 0.0)
    d1_ref[...] = jnp.dot(h, w02_ref[...],
                          preferred_element_type=jnp.float32) + b2_ref[...]
    y2_ref[...] = jnp.dot(h, w12_ref[...],
                          preferred_element_type=jnp.float32) * dis


def _tc_b(d0, p1, dis, W0_2, W1_2p, b2):
    return pl.pallas_call(
        _tc_b_body,
        grid=(_G,),
        in_specs=[
            pl.BlockSpec((_R, HID), lambda j: (j, 0)),
            pl.BlockSpec((NC, _R, HID), lambda j: (0, j, 0)),
            pl.BlockSpec((_R, 1), lambda j: (j, 0)),
            pl.BlockSpec((HID, N_CLASSES), lambda j: (0, 0)),
            pl.BlockSpec((HID, HID), lambda j: (0, 0)),
            pl.BlockSpec((1, N_CLASSES), lambda j: (0, 0)),
        ],
        out_specs=[
            pl.BlockSpec((_R, N_CLASSES), lambda j: (j, 0)),
            pl.BlockSpec((_R, HID), lambda j: (j, 0)),
        ],
        out_shape=[
            jax.ShapeDtypeStruct((NP, N_CLASSES), jnp.float32),
            jax.ShapeDtypeStruct((NP, HID), jnp.float32),
        ],
    )(d0, p1, dis, W0_2, W1_2p, b2)


def _tc_c_body(d1_ref, p2_ref, dis_ref, out_ref):
    p = p2_ref[0] + p2_ref[1]
    out_ref[...] = d1_ref[...] - dis_ref[...] * p[:, :N_CLASSES]


def _tc_c(d1, p2, dis):
    return pl.pallas_call(
        _tc_c_body,
        grid=(_G,),
        in_specs=[
            pl.BlockSpec((_R, N_CLASSES), lambda j: (j, 0)),
            pl.BlockSpec((NC, _R, HID), lambda j: (0, j, 0)),
            pl.BlockSpec((_R, 1), lambda j: (j, 0)),
        ],
        out_specs=pl.BlockSpec((_R, N_CLASSES), lambda j: (j, 0)),
        out_shape=jax.ShapeDtypeStruct((NP, N_CLASSES), jnp.float32),
    )(d1, p2, dis)


def kernel(x, adj, W0_1, W1_1, b1, W0_2, W1_2, b2):
    row = adj[0].astype(jnp.int32)
    col = adj[1].astype(jnp.int32)
    pad = jnp.full((EPAD - E,), NP - 1, jnp.int32)
    row3 = jnp.concatenate([row, pad]).reshape(NW, CH, K)
    col3 = jnp.concatenate([col, pad]).reshape(NW, CH, K)

    x_p = jnp.pad(x, ((0, NP - N), (0, 0)))
    W1_2p = jnp.pad(W1_2, ((0, 0), (0, HID - N_CLASSES)))
    b1r = b1.reshape(1, HID)
    b2r = b2.reshape(1, N_CLASSES)

    prop64 = _make_prop(HID)
    degp = _get_deg_kernel()(row3)
    d0, y1, dis = _tc_a(x_p, W0_1, W1_1, b1r, degp)
    p1 = prop64(y1, row3, col3)
    d1, y2 = _tc_b(d0, p1, dis, W0_2, W1_2p, b2r)
    p2 = prop64(y2, row3, col3)
    out = _tc_c(d1, p2, dis)
    return out[:N]


# speedup vs baseline: 13.9050x; 1.1524x over previous
import functools

import jax
import jax.numpy as jnp
from jax import lax
from jax.experimental import pallas as pl
from jax.experimental.pallas import tpu as pltpu
from jax.experimental.pallas import tpu_sc as plsc

N = 10000
E = 320000
F_IN = 128
HID = 64
N_CLASSES = 40

NC = 2
NS = 16
NW = NC * NS
NP = 10240
TILE_N = NP // NS
K = 128
CH = 80
EPT = CH * K
EPAD = EPT * NW

ROWS_PER_BOUNCE = 32
N_BOUNCE = TILE_N // ROWS_PER_BOUNCE
NBUF = 8

@functools.cache
def _get_mesh():
    return plsc.VectorSubcoreMesh(core_axis_name="c", subcore_axis_name="s",
                                  num_cores=NC, num_subcores=NS)


@functools.cache
def _get_deg_kernel():
    return functools.partial(
        pl.kernel,
        out_type=jax.ShapeDtypeStruct((NC, NP), jnp.float32),
        mesh=_get_mesh(),
        scratch_types=[
            pltpu.VMEM((CH, K), jnp.int32),
            pltpu.VMEM((K,), jnp.float32),
            pltpu.VMEM((TILE_N,), jnp.float32),
            pltpu.VMEM_SHARED((NP,), jnp.float32),
        ],
    )(_deg_body)


def _deg_body(row_hbm, out_hbm, rowv, ones, zbuf, acc):
    cid = lax.axis_index("c")
    sid = lax.axis_index("s")
    wid = sid * NC + cid

    pltpu.sync_copy(row_hbm.at[wid], rowv)

    def fill_ones(i, _):
        ones[pl.ds(i * 16, 16)] = jnp.full((16,), 1.0, jnp.float32)
        return 0

    lax.fori_loop(0, K // 16, fill_ones, 0)

    def fill_zero(i, _):
        zbuf[pl.ds(i * 16, 16)] = jnp.zeros((16,), jnp.float32)
        return 0

    lax.fori_loop(0, TILE_N // 16, fill_zero, 0)

    pltpu.sync_copy(zbuf, acc.at[pl.ds(sid * TILE_N, TILE_N)])
    plsc.subcore_barrier()

    def chunk(ci, _):
        pltpu.sync_copy(ones, acc.at[rowv.at[ci]], add=True)
        return 0

    lax.fori_loop(0, CH, chunk, 0)
    plsc.subcore_barrier()

    sl = pl.ds(sid * TILE_N, TILE_N)
    pltpu.sync_copy(acc.at[sl], zbuf)
    pltpu.sync_copy(zbuf, out_hbm.at[cid, sl])


@functools.cache
def _make_prop(D):
    @functools.partial(
        pl.kernel,
        out_type=jax.ShapeDtypeStruct((NC, NP, D), jnp.float32),
        mesh=_get_mesh(),
        scratch_types=[
            pltpu.VMEM((CH, K), jnp.int32),
            pltpu.VMEM((CH, K), jnp.int32),
            pltpu.VMEM((NBUF, K, D), jnp.float32),
            pltpu.VMEM((ROWS_PER_BOUNCE, D), jnp.float32),
            pltpu.VMEM_SHARED((NP, D), jnp.float32),
            [pltpu.SemaphoreType.DMA] * NBUF,
            [pltpu.SemaphoreType.DMA] * NBUF,
        ],
        compiler_params=pltpu.CompilerParams(use_tc_tiling_on_sc=False),
    )
    def _prop(y_hbm, row_hbm, col_hbm, out_hbm, rowv, colv, bufs, zbuf, acc,
              gsems, ssems):
        cid = lax.axis_index("c")
        sid = lax.axis_index("s")
        wid = sid * NC + cid

        pltpu.sync_copy(row_hbm.at[wid], rowv)
        pltpu.sync_copy(col_hbm.at[wid], colv)

        def fill_zero(i, _):
            r = i // (D // 16)
            c = (i % (D // 16)) * 16
            zbuf[r, pl.ds(c, 16)] = jnp.zeros((16,), jnp.float32)
            return 0

        lax.fori_loop(0, ROWS_PER_BOUNCE * (D // 16), fill_zero, 0)

        for b in range(N_BOUNCE):
            pltpu.sync_copy(
                zbuf,
                acc.at[pl.ds(sid * TILE_N + b * ROWS_PER_BOUNCE,
                             ROWS_PER_BOUNCE), :])
        plsc.subcore_barrier()

        for b in range(NBUF):
            pltpu.async_copy(y_hbm.at[rowv.at[b]], bufs.at[b], gsems[b])

        def slot(g, b, prefetch):
            ci = g * NBUF + b
            pltpu.make_async_copy(y_hbm.at[rowv.at[ci]], bufs.at[b],
                                  gsems[b]).wait()
            pltpu.async_copy(bufs.at[b], acc.at[colv.at[ci]], ssems[b],
                             add=True)
            pltpu.make_async_copy(bufs.at[b], acc.at[colv.at[ci]],
                                  ssems[b]).wait()
            if prefetch:
                pltpu.async_copy(y_hbm.at[rowv.at[ci + NBUF]], bufs.at[b],
                                 gsems[b])

        def group(g, _):
            for b in range(NBUF):
                slot(g, b, True)
            return 0

        lax.fori_loop(0, CH // NBUF - 1, group, 0)
        for b in range(NBUF):
            slot(CH // NBUF - 1, b, False)
        plsc.subcore_barrier()

        for b in range(N_BOUNCE):
            sl = pl.ds(sid * TILE_N + b * ROWS_PER_BOUNCE, ROWS_PER_BOUNCE)
            pltpu.sync_copy(acc.at[sl, :], zbuf)
            pltpu.sync_copy(zbuf, out_hbm.at[cid].at[sl, :])

    return _prop


_R = 1024
_G = NP // _R


def _tc_a_body(x_ref, w0_ref, w1_ref, b1_ref, deg_ref,
               d0_ref, y1_ref, dis_ref):
    deg = deg_ref[0, :] + deg_ref[1, :]
    dis = jnp.where(deg > 0, lax.rsqrt(deg), 0.0)[:, None]
    xb = x_ref[...]
    d0_ref[...] = jnp.dot(xb, w0_ref[...],
                          preferred_element_type=jnp.float32) + b1_ref[...]
    y1_ref[...] = jnp.dot(xb, w1_ref[...],
                          preferred_element_type=jnp.float32) * dis
    dis_ref[...] = dis


def _tc_a(x_p, W0_1, W1_1, b1, degp):
    return pl.pallas_call(
        _tc_a_body,
        grid=(_G,),
        in_specs=[
            pl.BlockSpec((_R, F_IN), lambda j: (j, 0)),
            pl.BlockSpec((F_IN, HID), lambda j: (0, 0)),
            pl.BlockSpec((F_IN, HID), lambda j: (0, 0)),
            pl.BlockSpec((1, HID), lambda j: (0, 0)),
            pl.BlockSpec((NC, _R), lambda j: (0, j)),
        ],
        out_specs=[
            pl.BlockSpec((_R, HID), lambda j: (j, 0)),
            pl.BlockSpec((_R, HID), lambda j: (j, 0)),
            pl.BlockSpec((_R, 1), lambda j: (j, 0)),
        ],
        out_shape=[
            jax.ShapeDtypeStruct((NP, HID), jnp.float32),
            jax.ShapeDtypeStruct((NP, HID), jnp.float32),
            jax.ShapeDtypeStruct((NP, 1), jnp.float32),
        ],
    )(x_p, W0_1, W1_1, b1, degp)


def _tc_b_body(d0_ref, p1_ref, dis_ref, w02_ref, w12_ref, b2_ref,
               d1_ref, y2_ref):
    p = p1_ref[0] + p1_ref[1]
    dis = dis_ref[...]
    h = jnp.maximum(d0_ref[...] - dis * p, 0.0)
    d1_ref[...] = jnp.dot(h, w02_ref[...],
                          preferred_element_type=jnp.float32) + b2_ref[...]
    y2_ref[...] = jnp.dot(h, w12_ref[...],
                          preferred_element_type=jnp.float32) * dis


def _tc_b(d0, p1, dis, W0_2, W1_2p, b2):
    return pl.pallas_call(
        _tc_b_body,
        grid=(_G,),
        in_specs=[
            pl.BlockSpec((_R, HID), lambda j: (j, 0)),
            pl.BlockSpec((NC, _R, HID), lambda j: (0, j, 0)),
            pl.BlockSpec((_R, 1), lambda j: (j, 0)),
            pl.BlockSpec((HID, N_CLASSES), lambda j: (0, 0)),
            pl.BlockSpec((HID, HID), lambda j: (0, 0)),
            pl.BlockSpec((1, N_CLASSES), lambda j: (0, 0)),
        ],
        out_specs=[
            pl.BlockSpec((_R, N_CLASSES), lambda j: (j, 0)),
            pl.BlockSpec((_R, HID), lambda j: (j, 0)),
        ],
        out_shape=[
            jax.ShapeDtypeStruct((NP, N_CLASSES), jnp.float32),
            jax.ShapeDtypeStruct((NP, HID), jnp.float32),
        ],
    )(d0, p1, dis, W0_2, W1_2p, b2)


def _tc_c_body(d1_ref, p2_ref, dis_ref, out_ref):
    p = p2_ref[0] + p2_ref[1]
    out_ref[...] = d1_ref[...] - dis_ref[...] * p[:, :N_CLASSES]


def _tc_c(d1, p2, dis):
    return pl.pallas_call(
        _tc_c_body,
        grid=(_G,),
        in_specs=[
            pl.BlockSpec((_R, N_CLASSES), lambda j: (j, 0)),
            pl.BlockSpec((NC, _R, HID), lambda j: (0, j, 0)),
            pl.BlockSpec((_R, 1), lambda j: (j, 0)),
        ],
        out_specs=pl.BlockSpec((_R, N_CLASSES), lambda j: (j, 0)),
        out_shape=jax.ShapeDtypeStruct((NP, N_CLASSES), jnp.float32),
    )(d1, p2, dis)


def kernel(x, adj, W0_1, W1_1, b1, W0_2, W1_2, b2):
    row = adj[0].astype(jnp.int32)
    col = adj[1].astype(jnp.int32)
    pad = jnp.full((EPAD - E,), NP - 1, jnp.int32)
    row3 = jnp.concatenate([row, pad]).reshape(NW, CH, K)
    col3 = jnp.concatenate([col, pad]).reshape(NW, CH, K)

    x_p = jnp.pad(x, ((0, NP - N), (0, 0)))
    W1_2p = jnp.pad(W1_2, ((0, 0), (0, HID - N_CLASSES)))
    b1r = b1.reshape(1, HID)
    b2r = b2.reshape(1, N_CLASSES)

    prop64 = _make_prop(HID)
    degp = _get_deg_kernel()(row3)
    d0, y1, dis = _tc_a(x_p, W0_1, W1_1, b1r, degp)
    p1 = prop64(y1, row3, col3)
    d1, y2 = _tc_b(d0, p1, dis, W0_2, W1_2p, b2r)
    p2 = prop64(y2, row3, col3)
    out = _tc_c(d1, p2, dis)
    return out[:N]
